# Initial kernel scaffold; baseline (speedup 1.0000x reference)
import jax
import jax.numpy as jnp
from jax.experimental import pallas as pl


def kernel(x):
    raise NotImplementedError("write your pallas kernel here")


# speedup vs baseline: 27.2104x; 27.2104x over previous
import functools

import jax
import jax.numpy as jnp
import numpy as np
from jax.experimental import pallas as pl

SPARSITY = 0.95
ROW_BLOCK = 256

_INT_MIN = np.int32(-(2**31))


def _bit_const(b: int):
    return np.int32(-(2**31)) if b == 31 else np.int32(1 << b)


def _topk_mask_kernel(x_ref, o_ref, *, k: int):
    xb = x_ref[...]
    s = jax.lax.bitcast_convert_type(xb, jnp.int32)
    v = jnp.where(s < 0, s ^ np.int32(0x7FFFFFFF), s)
    pu = jnp.zeros((xb.shape[0], 1), jnp.int32)
    for b in range(31, -1, -1):
        t = pu | _bit_const(b)
        pv = t ^ _INT_MIN
        cnt = jnp.sum((v >= pv).astype(jnp.int32), axis=1, keepdims=True)
        pu = jnp.where(cnt >= k, t, pu)
    thr = pu ^ _INT_MIN
    o_ref[...] = jnp.where(v >= thr, xb, jnp.float32(0.0))


def kernel(x):
    d = x.shape[-1]
    k = max(1, int(d * (1.0 - SPARSITY)))
    flat = x.reshape(-1, d)
    rows = flat.shape[0]
    out = pl.pallas_call(
        functools.partial(_topk_mask_kernel, k=k),
        grid=(rows // ROW_BLOCK,),
        in_specs=[pl.BlockSpec((ROW_BLOCK, d), lambda i: (i, 0))],
        out_specs=pl.BlockSpec((ROW_BLOCK, d), lambda i: (i, 0)),
        out_shape=jax.ShapeDtypeStruct((rows, d), jnp.float32),
    )(flat)
    return out.reshape(x.shape)


# speedup vs baseline: 27.2248x; 1.0005x over previous
import functools

import jax
import jax.numpy as jnp
import numpy as np
from jax.experimental import pallas as pl
from jax.experimental.pallas import tpu as pltpu

SPARSITY = 0.95
ROW_BLOCK = 256

_INT_MIN = np.int32(-(2**31))


def _bit_const(b: int):
    return np.int32(-(2**31)) if b == 31 else np.int32(1 << b)


def _topk_mask_kernel(x_ref, o_ref, *, k: int):
    xb = x_ref[...]
    s = jax.lax.bitcast_convert_type(xb, jnp.int32)
    v = jnp.where(s < 0, s ^ np.int32(0x7FFFFFFF), s)
    pu = jnp.zeros((xb.shape[0], 1), jnp.int32)
    for b in range(31, -1, -1):
        t = pu | _bit_const(b)
        pv = t ^ _INT_MIN
        cnt = jnp.sum((v >= pv).astype(jnp.int32), axis=1, keepdims=True)
        pu = jnp.where(cnt >= k, t, pu)
    thr = pu ^ _INT_MIN
    o_ref[...] = jnp.where(v >= thr, xb, jnp.float32(0.0))


def kernel(x):
    d = x.shape[-1]
    k = max(1, int(d * (1.0 - SPARSITY)))
    flat = x.reshape(-1, d)
    rows = flat.shape[0]
    out = pl.pallas_call(
        functools.partial(_topk_mask_kernel, k=k),
        grid=(rows // ROW_BLOCK,),
        in_specs=[pl.BlockSpec((ROW_BLOCK, d), lambda i: (i, 0))],
        out_specs=pl.BlockSpec((ROW_BLOCK, d), lambda i: (i, 0)),
        out_shape=jax.ShapeDtypeStruct((rows, d), jnp.float32),
        compiler_params=pltpu.CompilerParams(
            dimension_semantics=("parallel",),
        ),
    )(flat)
    return out.reshape(x.shape)


# speedup vs baseline: 44.9240x; 1.6501x over previous
import functools

import jax
import jax.numpy as jnp
import numpy as np
from jax.experimental import pallas as pl
from jax.experimental.pallas import tpu as pltpu

SPARSITY = 0.95
ROW_BLOCK = 256

_I16_MIN = np.int16(-(2**15))


def _count16(mask):
    c = mask.astype(jnp.int16)
    w = c.shape[1]
    while w > 256:
        c = c[:, : w // 2] + c[:, w // 2 :]
        w //= 2
    return jnp.sum(c.astype(jnp.int32), axis=1, keepdims=True)


def _topk_mask_kernel(x_ref, o_ref, *, k: int):
    xb = x_ref[...]
    s = jax.lax.bitcast_convert_type(xb, jnp.int32)
    v = jnp.where(s < 0, s ^ np.int32(0x7FFFFFFF), s)

    hi = (v >> 16).astype(jnp.int16)
    pu = jnp.zeros((xb.shape[0], 1), jnp.int32)
    for b in range(15, -1, -1):
        t = pu | np.int32(1 << b)
        pv = (t - 32768).astype(jnp.int16)
        cnt = _count16(hi >= pv)
        pu = jnp.where(cnt >= k, t, pu)
    thr_hi = (pu - 32768).astype(jnp.int16)

    c_gt = _count16(hi > thr_hi)
    lo_s = ((v & np.int32(0xFFFF)) - 32768).astype(jnp.int16)
    w = jnp.where(hi == thr_hi, lo_s, _I16_MIN)

    pl2 = jnp.zeros((xb.shape[0], 1), jnp.int32)
    for b in range(15, -1, -1):
        t = pl2 | np.int32(1 << b)
        pv = (t - 32768).astype(jnp.int16)
        cnt = _count16(w >= pv)
        pl2 = jnp.where(c_gt + cnt >= k, t, pl2)

    thr = ((pu - 32768) << 16) | pl2
    o_ref[...] = jnp.where(v >= thr, xb, jnp.float32(0.0))


def kernel(x):
    d = x.shape[-1]
    k = max(1, int(d * (1.0 - SPARSITY)))
    flat = x.reshape(-1, d)
    rows = flat.shape[0]
    out = pl.pallas_call(
        functools.partial(_topk_mask_kernel, k=k),
        grid=(rows // ROW_BLOCK,),
        in_specs=[pl.BlockSpec((ROW_BLOCK, d), lambda i: (i, 0))],
        out_specs=pl.BlockSpec((ROW_BLOCK, d), lambda i: (i, 0)),
        out_shape=jax.ShapeDtypeStruct((rows, d), jnp.float32),
        compiler_params=pltpu.CompilerParams(
            dimension_semantics=("parallel",),
        ),
    )(flat)
    return out.reshape(x.shape)


# speedup vs baseline: 46.2471x; 1.0295x over previous
import functools

import jax
import jax.numpy as jnp
import numpy as np
from jax.experimental import pallas as pl
from jax.experimental.pallas import tpu as pltpu

SPARSITY = 0.95
ROW_BLOCK = 256

_I16_MIN = np.int16(-(2**15))


def _count16(mask):
    c = mask.astype(jnp.int16)
    w = c.shape[1]
    while w > 128:
        c = c[:, : w // 2] + c[:, w // 2 :]
        w //= 2
    return jnp.sum(c.astype(jnp.int32), axis=1, keepdims=True)


def _topk_mask_kernel(x_ref, o_ref, *, k: int):
    xb = x_ref[...]
    s = jax.lax.bitcast_convert_type(xb, jnp.int32)

    h0 = (s >> 16).astype(jnp.int16)
    neg = h0 < 0
    hi = jnp.where(neg, h0 ^ np.int16(0x7FFF), h0)
    l0 = ((s & np.int32(0xFFFF)) - 32768).astype(jnp.int16)
    lo = jnp.where(neg, l0 ^ np.int16(-1), l0)

    pu = jnp.zeros((xb.shape[0], 1), jnp.int32)
    for b in range(15, -1, -1):
        t = pu | np.int32(1 << b)
        pv = (t - 32768).astype(jnp.int16)
        cnt = _count16(hi >= pv)
        pu = jnp.where(cnt >= k, t, pu)
    thr_hi = (pu - 32768).astype(jnp.int16)

    c_gt = _count16(hi > thr_hi)
    w = jnp.where(hi == thr_hi, lo, _I16_MIN)

    pl2 = jnp.zeros((xb.shape[0], 1), jnp.int32)
    for b in range(15, -1, -1):
        t = pl2 | np.int32(1 << b)
        pv = (t - 32768).astype(jnp.int16)
        cnt = _count16(w >= pv)
        pl2 = jnp.where(c_gt + cnt >= k, t, pl2)

    thr_v = ((pu - 32768) << 16) | pl2
    thr_s = jnp.where(thr_v < 0, thr_v ^ np.int32(0x7FFFFFFF), thr_v)
    thr_f = jax.lax.bitcast_convert_type(thr_s, jnp.float32)
    o_ref[...] = jnp.where(xb >= thr_f, xb, jnp.float32(0.0))


def kernel(x):
    d = x.shape[-1]
    k = max(1, int(d * (1.0 - SPARSITY)))
    flat = x.reshape(-1, d)
    rows = flat.shape[0]
    out = pl.pallas_call(
        functools.partial(_topk_mask_kernel, k=k),
        grid=(rows // ROW_BLOCK,),
        in_specs=[pl.BlockSpec((ROW_BLOCK, d), lambda i: (i, 0))],
        out_specs=pl.BlockSpec((ROW_BLOCK, d), lambda i: (i, 0)),
        out_shape=jax.ShapeDtypeStruct((rows, d), jnp.float32),
        compiler_params=pltpu.CompilerParams(
            dimension_semantics=("parallel",),
        ),
    )(flat)
    return out.reshape(x.shape)
